# baseline (speedup 1.0000x reference)
import functools

import jax
import jax.numpy as jnp
from jax import lax
from jax.experimental import pallas as pl
from jax.experimental.pallas import tpu as pltpu
from jax.experimental.pallas import tpu_sc as plsc

NC = 2
NS = 16
L = 16
NW = NC * NS

ROWS = 16384
COLS = 200
ROWS_PER_W = ROWS // NW
RCHUNK = 64
NCHUNK = ROWS_PER_W // RCHUNK

_COL_OFFS = tuple(range(0, COLS - L - 7, L))
TAIL = COLS - L

_mesh = plsc.VectorSubcoreMesh(core_axis_name="c", subcore_axis_name="s")


@functools.partial(
    pl.kernel,
    mesh=_mesh,
    out_type=jax.ShapeDtypeStruct((ROWS, COLS), jnp.float32),
    scratch_types=[
        pltpu.VMEM((2, L), jnp.float32),
        pltpu.VMEM((2, RCHUNK, COLS), jnp.int32),
        pltpu.VMEM((2, RCHUNK, COLS), jnp.float32),
        [pltpu.SemaphoreType.DMA] * 4,
    ],
)
def _emb_lookup(idx_hbm, w_hbm, out_hbm, w_v, idx_v, out_v, sems):
    wid = lax.axis_index("s") * NC + lax.axis_index("c")
    base = wid * ROWS_PER_W
    s_im, s_om = sems[0:2], sems[2:4]

    pltpu.sync_copy(w_hbm, w_v)
    w0 = w_v[0, :]
    d = w_v[1, :] - w0

    def rows_of(c):
        return pl.ds(base + c * RCHUNK, RCHUNK)

    def in_main(c):
        b = c % 2
        return pltpu.make_async_copy(
            idx_hbm.at[rows_of(c), :], idx_v.at[b], s_im[b])

    def out_main(c):
        b = c % 2
        return pltpu.make_async_copy(
            out_v.at[b], out_hbm.at[rows_of(c), :], s_om[b])

    tail_off = wid - wid + (COLS - 8)

    for c in range(min(2, NCHUNK)):
        in_main(c).start()
    for c in range(NCHUNK):
        b = c % 2
        in_main(c).wait()
        if c >= 2:
            out_main(c - 2).wait()

        @plsc.parallel_loop(0, RCHUNK, step=1, unroll=2)
        def _main_loop(r):
            for off in _COL_OFFS:
                x = idx_v[b, r, pl.ds(off, L)]
                out_v[b, r, pl.ds(off, L)] = w0 + d * x.astype(jnp.float32)
            xt = idx_v[b, r, pl.ds(tail_off, L)]
            out_v[b, r, pl.ds(tail_off, L)] = w0 + d * xt.astype(jnp.float32)

        out_main(c).start()
        if c + 2 < NCHUNK:
            in_main(c + 2).start()
    out_main(NCHUNK - 2).wait()
    out_main(NCHUNK - 1).wait()


def kernel(idx, weight):
    wb = jnp.broadcast_to(weight.astype(jnp.float32), (2, L))
    out = _emb_lookup(idx, wb)
    return out.reshape(ROWS, COLS, 1)


# speedup vs baseline: 1.1312x; 1.1312x over previous
import functools

import jax
import jax.numpy as jnp
from jax import lax
from jax.experimental import pallas as pl
from jax.experimental.pallas import tpu as pltpu
from jax.experimental.pallas import tpu_sc as plsc

NC = 2
NS = 16
L = 16
NW = NC * NS

ROWS = 16384
COLS = 200
JT = COLS // 8
IHI = ROWS // 128
UNITS = COLS * 4
PER_W = UNITS // NW
UR = 32

_mesh = plsc.VectorSubcoreMesh(core_axis_name="c", subcore_axis_name="s")


@functools.partial(
    pl.kernel,
    mesh=_mesh,
    out_type=jax.ShapeDtypeStruct((COLS * IHI, 128), jnp.float32),
    scratch_types=[
        pltpu.VMEM((2, L), jnp.float32),
        pltpu.VMEM((2, UR, 128), jnp.int32),
        pltpu.VMEM((2, UR, 128), jnp.float32),
        [pltpu.SemaphoreType.DMA] * 4,
    ],
)
def _emb_lookup(idx_hbm, w_hbm, out_hbm, w_v, idx_v, out_v, sems):
    wid = lax.axis_index("s") * NC + lax.axis_index("c")
    u0 = wid * PER_W
    s_in, s_out = sems[0:2], sems[2:4]

    pltpu.sync_copy(w_hbm, w_v)
    w0 = w_v[0, :]
    d = w_v[1, :] - w0

    def in_copy(k):
        b = k % 2
        u = u0 + k
        j = u // 4
        q = u % 4
        return pltpu.make_async_copy(
            idx_hbm.at[j // 8, pl.ds(q * UR, UR), pl.ds((j % 8) * 128, 128)],
            idx_v.at[b], s_in[b])

    def out_copy(k):
        b = k % 2
        u = u0 + k
        return pltpu.make_async_copy(
            out_v.at[b], out_hbm.at[pl.ds(u * UR, UR), :], s_out[b])

    in_copy(0).start()
    in_copy(1).start()
    for k in range(PER_W):
        b = k % 2
        in_copy(k).wait()
        if k >= 2:
            out_copy(k - 2).wait()

        @plsc.parallel_loop(0, UR, step=1, unroll=2)
        def _unit_loop(r):
            for off in range(0, 128, L):
                x = idx_v[b, r, pl.ds(off, L)]
                out_v[b, r, pl.ds(off, L)] = w0 + d * x.astype(jnp.float32)

        out_copy(k).start()
        if k + 2 < PER_W:
            in_copy(k + 2).start()
    out_copy(PER_W - 2).wait()
    out_copy(PER_W - 1).wait()


def kernel(idx, weight):
    idx3 = (idx.T.reshape(JT, 8, IHI, 128)
            .transpose(0, 2, 1, 3).reshape(JT, IHI, 8 * 128))
    wb = jnp.broadcast_to(weight.astype(jnp.float32), (2, L))
    out2 = _emb_lookup(idx3, wb)
    return (out2.reshape(COLS, IHI, 128).transpose(1, 2, 0)
            .reshape(ROWS, COLS, 1))


# speedup vs baseline: 1.3445x; 1.1886x over previous
import functools

import jax
import jax.numpy as jnp
from jax import lax
from jax.experimental import pallas as pl
from jax.experimental.pallas import tpu as pltpu
from jax.experimental.pallas import tpu_sc as plsc

NC = 2
NS = 16
L = 16
NW = NC * NS

ROWS = 16384
COLS = 200
JT = COLS // 8
IHI = ROWS // 128
SLABS = JT * IHI
NB = 5
NT = 5

_mesh = plsc.VectorSubcoreMesh(core_axis_name="c", subcore_axis_name="s")


@functools.partial(
    pl.kernel,
    mesh=_mesh,
    out_type=jax.ShapeDtypeStruct((SLABS, 8, 128), jnp.float32),
    scratch_types=[
        pltpu.VMEM((2, L), jnp.float32),
        pltpu.VMEM((NB, 4, 8, 128), jnp.int32),
        pltpu.VMEM((NB, 4, 8, 128), jnp.float32),
        [pltpu.SemaphoreType.DMA] * (2 * NB),
    ],
)
def _emb_lookup(idx_hbm, w_hbm, out_hbm, w_v, idx_v, out_v, sems):
    wid = lax.axis_index("s") * NC + lax.axis_index("c")
    u0 = wid * NB * NT
    s_in, s_out = sems[:NB], sems[NB:]

    pltpu.sync_copy(w_hbm, w_v)
    w0 = w_v[0, :]
    d = w_v[1, :] - w0

    def in_copy(u, m):
        return pltpu.make_async_copy(
            idx_hbm.at[pl.ds(u * 4, 4), :, :], idx_v.at[m], s_in[m])

    def out_copy(u, m):
        return pltpu.make_async_copy(
            out_v.at[m], out_hbm.at[pl.ds(u * 4, 4), :, :], s_out[m])

    def round_body(t, _):
        for m in range(NB):
            in_copy(u0 + t * NB + m, m).start()
        for m in range(NB):
            u = u0 + t * NB + m
            in_copy(u, m).wait()

            @pl.when(t > 0)
            def _():
                out_copy(u - NB, m).wait()

            @plsc.parallel_loop(0, 4, step=1)
            def _unit(r):
                for jj in range(8):
                    for off in range(0, 128, L):
                        x = idx_v[m, r, jj, pl.ds(off, L)]
                        out_v[m, r, jj, pl.ds(off, L)] = (
                            w0 + d * x.astype(jnp.float32))

            out_copy(u, m).start()
        return 0

    lax.fori_loop(0, NT, round_body, 0)
    for m in range(NB):
        out_copy(u0 + (NT - 1) * NB + m, m).wait()


def kernel(idx, weight):
    idx3 = (idx.T.reshape(JT, 8, IHI, 128)
            .transpose(0, 2, 1, 3).reshape(SLABS, 8, 128))
    wb = jnp.broadcast_to(weight.astype(jnp.float32), (2, L))
    out3 = _emb_lookup(idx3, wb)
    return (out3.reshape(JT, IHI, 8, 128).transpose(0, 2, 1, 3)
            .reshape(COLS, ROWS).T.reshape(ROWS, COLS, 1))
